# baseline (speedup 1.0000x reference)
import jax
import jax.numpy as jnp
from jax.experimental import pallas as pl

_NUM_RAYS = 1024
_TB = 128
_W = _NUM_RAYS * 3


def _body(idx_ref, tbl_ref, out_ld_ref, out_li_ref):
    tb = idx_ref.shape[-1]
    nl = tbl_ref.shape[0]
    idx = idx_ref[0, 0, :]
    onehot = (jax.lax.broadcasted_iota(jnp.int32, (tb, nl), 1) == idx[:, None])
    vals = jax.lax.dot_general(
        onehot.astype(jnp.float32), tbl_ref[...],
        (((1,), (0,)), ((), ())), preferred_element_type=jnp.float32)
    x = vals[:, 0:1]
    y = vals[:, 1:2]
    z = -jnp.abs(vals[:, 2:3])
    inten = jnp.abs(vals[:, 3:4])
    inv = 1.0 / jnp.maximum(jnp.sqrt(x * x + y * y + z * z), 1e-12)
    c = jax.lax.broadcasted_iota(jnp.int32, (tb, _W), 1) % 3
    xs = jnp.broadcast_to(x * inv, (tb, _W))
    ys = jnp.broadcast_to(y * inv, (tb, _W))
    zs = jnp.broadcast_to(z * inv, (tb, _W))
    out_ld_ref[...] = jnp.where(c == 0, xs, jnp.where(c == 1, ys, zs))
    out_li_ref[...] = jnp.broadcast_to(inten, (tb, _W))


def kernel(idx, light_direction_xy, light_direction_z, light_intensity):
    b = idx.shape[0]
    tbl = jnp.concatenate(
        [light_direction_xy, light_direction_z, light_intensity], axis=1)
    grid = b // _TB
    idx3 = idx.reshape(grid, 1, _TB)
    out_ld, out_li = pl.pallas_call(
        _body,
        grid=(grid,),
        in_specs=[
            pl.BlockSpec((1, 1, _TB), lambda i: (i, 0, 0)),
            pl.BlockSpec(tbl.shape, lambda i: (0, 0)),
        ],
        out_specs=[
            pl.BlockSpec((_TB, _W), lambda i: (i, 0)),
            pl.BlockSpec((_TB, _W), lambda i: (i, 0)),
        ],
        out_shape=[
            jax.ShapeDtypeStruct((b, _W), jnp.float32),
            jax.ShapeDtypeStruct((b, _W), jnp.float32),
        ],
    )(idx3, tbl)
    return (out_ld.reshape(-1, 3), out_li.reshape(-1, 3))


# speedup vs baseline: 25.2830x; 25.2830x over previous
import jax
import jax.numpy as jnp
from jax.experimental import pallas as pl

_NUM_RAYS = 1024
_TR = 512


def _body(idx_ref, tbl_ref, out_ld_ref, out_li_ref):
    tr = idx_ref.shape[-1]
    nl = tbl_ref.shape[0]
    nb = pl.num_programs(0) // 3
    c = pl.program_id(0) // nb
    idx = idx_ref[0, 0, :]
    onehot = (jax.lax.broadcasted_iota(jnp.int32, (tr, nl), 1) == idx[:, None])
    vals = jax.lax.dot_general(
        onehot.astype(jnp.float32), tbl_ref[...],
        (((1,), (0,)), ((), ())), preferred_element_type=jnp.float32)
    x = vals[:, 0:1]
    y = vals[:, 1:2]
    z = -jnp.abs(vals[:, 2:3])
    inten = jnp.abs(vals[:, 3:4])
    n = jnp.maximum(jnp.sqrt(x * x + y * y + z * z), 1e-12)
    col = jnp.where(c == 0, x, jnp.where(c == 1, y, z)) / n
    out_ld_ref[...] = jnp.broadcast_to(col, (tr, _NUM_RAYS))
    out_li_ref[...] = jnp.broadcast_to(inten, (tr, _NUM_RAYS))


def kernel(idx, light_direction_xy, light_direction_z, light_intensity):
    b = idx.shape[0]
    tbl = jnp.concatenate(
        [light_direction_xy, light_direction_z, light_intensity], axis=1)
    nb = b // _TR
    grid = 3 * nb
    idx3 = idx.reshape(nb, 1, _TR)
    out_ld, out_li = pl.pallas_call(
        _body,
        grid=(grid,),
        in_specs=[
            pl.BlockSpec((1, 1, _TR), lambda i, nb=nb: (jax.lax.rem(i, nb), 0, 0)),
            pl.BlockSpec(tbl.shape, lambda i: (0, 0)),
        ],
        out_specs=[
            pl.BlockSpec((_TR, _NUM_RAYS), lambda i: (i, 0)),
            pl.BlockSpec((_TR, _NUM_RAYS), lambda i: (i, 0)),
        ],
        out_shape=[
            jax.ShapeDtypeStruct((3 * b, _NUM_RAYS), jnp.float32),
            jax.ShapeDtypeStruct((3 * b, _NUM_RAYS), jnp.float32),
        ],
    )(idx3, tbl)
    out_ld = out_ld.reshape(3, b * _NUM_RAYS).T
    out_li = out_li.reshape(3, b * _NUM_RAYS).T
    return (out_ld, out_li)


# speedup vs baseline: 25.4368x; 1.0061x over previous
import jax
import jax.numpy as jnp
from jax.experimental import pallas as pl

_NUM_RAYS = 1024
_TR = 1024


def _body(idx_ref, tbl_ref, out_ld_ref, out_li_ref):
    tr = idx_ref.shape[-1]
    nl = tbl_ref.shape[0]
    nb = pl.num_programs(0) // 3
    c = pl.program_id(0) // nb
    idx = idx_ref[0, 0, :]
    onehot = (jax.lax.broadcasted_iota(jnp.int32, (tr, nl), 1) == idx[:, None])
    vals = jax.lax.dot_general(
        onehot.astype(jnp.float32), tbl_ref[...],
        (((1,), (0,)), ((), ())), preferred_element_type=jnp.float32)
    x = vals[:, 0:1]
    y = vals[:, 1:2]
    z = -jnp.abs(vals[:, 2:3])
    inten = jnp.abs(vals[:, 3:4])
    n = jnp.maximum(jnp.sqrt(x * x + y * y + z * z), 1e-12)
    col = jnp.where(c == 0, x, jnp.where(c == 1, y, z)) / n
    out_ld_ref[...] = jnp.broadcast_to(col, (tr, _NUM_RAYS))
    out_li_ref[...] = jnp.broadcast_to(inten, (tr, _NUM_RAYS))


def kernel(idx, light_direction_xy, light_direction_z, light_intensity):
    b = idx.shape[0]
    tbl = jnp.concatenate(
        [light_direction_xy, light_direction_z, light_intensity], axis=1)
    nb = b // _TR
    grid = 3 * nb
    idx3 = idx.reshape(nb, 1, _TR)
    out_ld, out_li = pl.pallas_call(
        _body,
        grid=(grid,),
        in_specs=[
            pl.BlockSpec((1, 1, _TR), lambda i, nb=nb: (jax.lax.rem(i, nb), 0, 0)),
            pl.BlockSpec(tbl.shape, lambda i: (0, 0)),
        ],
        out_specs=[
            pl.BlockSpec((_TR, _NUM_RAYS), lambda i: (i, 0)),
            pl.BlockSpec((_TR, _NUM_RAYS), lambda i: (i, 0)),
        ],
        out_shape=[
            jax.ShapeDtypeStruct((3 * b, _NUM_RAYS), jnp.float32),
            jax.ShapeDtypeStruct((3 * b, _NUM_RAYS), jnp.float32),
        ],
    )(idx3, tbl)
    out_ld = out_ld.reshape(3, b * _NUM_RAYS).T
    out_li = out_li.reshape(3, b * _NUM_RAYS).T
    return (out_ld, out_li)
